# baseline (speedup 1.0000x reference)
import functools

import jax
import jax.numpy as jnp
from jax.experimental import pallas as pl
from jax.experimental.pallas import tpu as pltpu

E = 8
K = 2
D = 768
H = 2048
N = 2048
TN = 256


def _gate_weights(scores):
    ids = jax.lax.broadcasted_iota(jnp.int32, scores.shape, 1)
    m1 = jnp.max(scores, axis=-1, keepdims=True)
    i1 = jnp.min(jnp.where(scores == m1, ids, E), axis=-1, keepdims=True)
    s2 = jnp.where(ids == i1, -jnp.inf, scores)
    m2 = jnp.max(s2, axis=-1, keepdims=True)
    i2 = jnp.min(jnp.where(s2 == m2, ids, E), axis=-1, keepdims=True)
    mask = (ids == i1) | (ids == i2)
    p = jnp.exp(scores - m1)
    p = p / jnp.sum(p, axis=-1, keepdims=True)
    w = jnp.where(mask, p, 0.0)
    return w / (jnp.sum(w, axis=-1, keepdims=True) + 1e-8)


def _moe_body(x_ref, Wg_ref, bg_ref, W1_ref, b1_ref, W2_ref, b2_ref,
              out_ref, acc_ref):
    e = pl.program_id(0)
    t = pl.program_id(1)
    xt = x_ref[pl.ds(t * TN, TN), :]

    scores = jnp.dot(xt, Wg_ref[...], preferred_element_type=jnp.float32)
    scores = scores + bg_ref[0]
    w = _gate_weights(scores)
    eids = jax.lax.broadcasted_iota(jnp.int32, w.shape, 1)
    we = jnp.sum(jnp.where(eids == e, w, 0.0), axis=-1, keepdims=True)

    h = jnp.dot(xt, W1_ref[0], preferred_element_type=jnp.float32) + b1_ref[0, 0]
    h = jnp.maximum(h, 0.0)
    y = jnp.dot(h, W2_ref[0], preferred_element_type=jnp.float32) + b2_ref[0, 0]
    y = we * y

    @pl.when(e == 0)
    def _():
        acc_ref[pl.ds(t * TN, TN), :] = y

    @pl.when(e > 0)
    def _():
        acc_ref[pl.ds(t * TN, TN), :] += y

    @pl.when(e == E - 1)
    def _():
        out_ref[...] = acc_ref[pl.ds(t * TN, TN), :]


@jax.jit
def kernel(x, W_gate, b_gate, W1, b1, W2, b2):
    grid = (E, N // TN)
    return pl.pallas_call(
        _moe_body,
        grid=grid,
        in_specs=[
            pl.BlockSpec((N, D), lambda e, t: (0, 0)),
            pl.BlockSpec((D, E), lambda e, t: (0, 0)),
            pl.BlockSpec((1, E), lambda e, t: (0, 0)),
            pl.BlockSpec((1, D, H), lambda e, t: (e, 0, 0)),
            pl.BlockSpec((1, 1, H), lambda e, t: (e, 0, 0)),
            pl.BlockSpec((1, H, D), lambda e, t: (e, 0, 0)),
            pl.BlockSpec((1, 1, D), lambda e, t: (e, 0, 0)),
        ],
        out_specs=pl.BlockSpec((TN, D), lambda e, t: (t, 0)),
        out_shape=jax.ShapeDtypeStruct((N, D), jnp.float32),
        scratch_shapes=[pltpu.VMEM((N, D), jnp.float32)],
        compiler_params=pltpu.CompilerParams(
            dimension_semantics=("arbitrary", "arbitrary"),
        ),
    )(x, W_gate, b_gate.reshape(1, E), W1, b1.reshape(E, 1, H), W2,
      b2.reshape(E, 1, D))
